# baseline (speedup 1.0000x reference)
import functools

import jax
import jax.numpy as jnp
import numpy as np
from jax.experimental import pallas as pl
from jax.experimental.pallas import tpu as pltpu

EMBED_DIM = 128
NUM_HEADS = 8
HEAD_DIM = EMBED_DIM // NUM_HEADS
SEQ = 32
NTOK = 750
TBLK = 25
ROWS = TBLK * SEQ


def _consts():
    half = HEAD_DIM // 2
    angle = 1.0 / 10000.0 ** np.linspace(0.0, 1.0, half)
    angle = np.repeat(angle, 2)
    angle_full = np.tile(angle, NUM_HEADS)
    idx = np.arange(SEQ, dtype=np.float64)
    sin = np.sin(idx[:, None] * angle_full[None, :]).astype(np.float32)
    cos = np.cos(idx[:, None] * angle_full[None, :]).astype(np.float32)

    P16 = np.zeros((HEAD_DIM, HEAD_DIM), np.float32)
    for o in range(half):
        P16[2 * o + 1, o] = -1.0
    for o in range(half, HEAD_DIM):
        P16[2 * (o - half), o] = 1.0
    P = np.zeros((EMBED_DIM, EMBED_DIM), np.float32)
    for h in range(NUM_HEADS):
        P[h * 16:(h + 1) * 16, h * 16:(h + 1) * 16] = P16

    decay = np.log(1.0 - 2.0 ** (-1.0 - 3.0 * np.arange(NUM_HEADS, dtype=np.float64) / NUM_HEADS))
    ij = np.abs(idx[:, None] - idx[None, :])
    mask_add = (decay[:, None, None] * ij[None]).reshape(NUM_HEADS * SEQ, SEQ).astype(np.float32)

    fm = np.zeros((NUM_HEADS, EMBED_DIM), np.float32)
    for h in range(NUM_HEADS):
        fm[h, h * 16:(h + 1) * 16] = 1.0
    return cos, sin, P, mask_add, fm


_COS, _SIN, _P, _MASK_ADD, _FM = _consts()


def _ln(x, g, b, eps=1e-5):
    mu = jnp.mean(x, axis=-1, keepdims=True)
    var = jnp.mean((x - mu) ** 2, axis=-1, keepdims=True)
    return (x - mu) * jax.lax.rsqrt(var + eps) * g + b


def _block_kernel(xp_ref, wq_ref, bq_ref, wk_ref, bk_ref, wv_ref, bv_ref,
                  g1_ref, be1_ref, w1_ref, b1_ref, w2_ref, b2_ref,
                  g2_ref, be2_ref, cos_ref, sin_ref, p_ref, mask_ref, fm_ref,
                  out_ref):
    xb = xp_ref[:]
    f32 = jnp.float32
    bf16 = jnp.bfloat16

    def mm(a, b):
        return jax.lax.dot_general(a.astype(bf16), b.astype(bf16),
                                   (((1,), (0,)), ((), ())),
                                   preferred_element_type=f32)

    q = mm(xb, wq_ref[:]) + bq_ref[:]
    k = mm(xb, wk_ref[:]) + bk_ref[:]
    v = mm(xb, wv_ref[:]) + bv_ref[:]

    cos = cos_ref[:]
    sin = sin_ref[:]
    P = p_ref[:]

    def rot(t):
        t3 = t.reshape(TBLK, SEQ, EMBED_DIM)
        tp3 = mm(t, P).reshape(TBLK, SEQ, EMBED_DIM)
        return (t3 * cos[None] + tp3 * sin[None]).reshape(ROWS, EMBED_DIM)

    q = rot(q)
    k = rot(k)

    fm = fm_ref[:]
    qm = (q.reshape(TBLK, 1, SEQ, EMBED_DIM) * fm[None, :, None, :]
          ).reshape(TBLK * NUM_HEADS * SEQ, EMBED_DIM)

    mask_add = mask_ref[:]
    outs = []
    for t in range(TBLK):
        qm_t = qm[t * NUM_HEADS * SEQ:(t + 1) * NUM_HEADS * SEQ]
        k_t = k[t * SEQ:(t + 1) * SEQ]
        v_t = v[t * SEQ:(t + 1) * SEQ]
        s = jax.lax.dot_general(qm_t.astype(bf16), k_t.astype(bf16),
                                (((1,), (1,)), ((), ())),
                                preferred_element_type=f32)
        s = s + mask_add
        m = jnp.max(s, axis=1, keepdims=True)
        e = jnp.exp(s - m)
        a = e / jnp.sum(e, axis=1, keepdims=True)
        o8 = jax.lax.dot_general(a.astype(bf16), v_t.astype(bf16),
                                 (((1,), (0,)), ((), ())),
                                 preferred_element_type=f32)
        o = jnp.sum(o8.reshape(NUM_HEADS, SEQ, EMBED_DIM) * fm[:, None, :],
                    axis=0)
        outs.append(o)
    attn = jnp.concatenate(outs, axis=0)

    y = _ln(attn + xb, g1_ref[:], be1_ref[:])
    h1 = jnp.maximum(mm(y, w1_ref[:]) + b1_ref[:], 0.0)
    ffn = mm(h1, w2_ref[:]) + b2_ref[:]
    out_ref[:] = _ln(ffn + y, g2_ref[:], be2_ref[:])


@jax.jit
def kernel(x, Wq, bq, Wk, bk, Wv, bv, g1, be1, W1, b1, W2, b2, g2, be2):
    B, N, C = x.shape
    xp = jnp.transpose(x, (1, 0, 2)).reshape(N * B, C)

    grid = N // TBLK
    full = lambda shape: pl.BlockSpec(shape, lambda i: (0,) * len(shape))
    out = pl.pallas_call(
        _block_kernel,
        grid=(grid,),
        in_specs=[
            pl.BlockSpec((ROWS, C), lambda i: (i, 0)),
            full((C, C)), full((1, C)),
            full((C, C)), full((1, C)),
            full((C, C)), full((1, C)),
            full((1, C)), full((1, C)),
            full((C, 4 * C)), full((1, 4 * C)),
            full((4 * C, C)), full((1, C)),
            full((1, C)), full((1, C)),
            full((SEQ, C)), full((SEQ, C)), full((C, C)),
            full((NUM_HEADS * SEQ, SEQ)), full((NUM_HEADS, C)),
        ],
        out_specs=pl.BlockSpec((ROWS, C), lambda i: (i, 0)),
        out_shape=jax.ShapeDtypeStruct((N * B, C), jnp.float32),
    )(xp, Wq, bq.reshape(1, C), Wk, bk.reshape(1, C), Wv, bv.reshape(1, C),
      g1.reshape(1, C), be1.reshape(1, C), W1, b1.reshape(1, 4 * C),
      W2, b2.reshape(1, C), g2.reshape(1, C), be2.reshape(1, C),
      jnp.asarray(_COS), jnp.asarray(_SIN), jnp.asarray(_P),
      jnp.asarray(_MASK_ADD), jnp.asarray(_FM))

    return out.reshape(N, B, C).transpose(1, 0, 2)


# speedup vs baseline: 1.4008x; 1.4008x over previous
import jax
import jax.numpy as jnp
import numpy as np
from jax.experimental import pallas as pl

EMBED_DIM = 128
NUM_HEADS = 8
HEAD_DIM = EMBED_DIM // NUM_HEADS
SEQ = 32
NTOK = 750
TBLK = 25
ROWS = TBLK * SEQ
HS = NUM_HEADS * SEQ


def _consts():
    half = HEAD_DIM // 2
    angle = 1.0 / 10000.0 ** np.linspace(0.0, 1.0, half)
    angle = np.repeat(angle, 2)
    angle_full = np.tile(angle, NUM_HEADS)
    idx = np.arange(SEQ, dtype=np.float64)
    sin = np.sin(idx[:, None] * angle_full[None, :])
    cos = np.cos(idx[:, None] * angle_full[None, :])

    P16 = np.zeros((HEAD_DIM, HEAD_DIM), np.float32)
    for o in range(half):
        P16[2 * o + 1, o] = -1.0
    for o in range(half, HEAD_DIM):
        P16[2 * (o - half), o] = 1.0
    P = np.zeros((EMBED_DIM, EMBED_DIM), np.float32)
    for h in range(NUM_HEADS):
        P[h * 16:(h + 1) * 16, h * 16:(h + 1) * 16] = P16

    decay = np.log(1.0 - 2.0 ** (-1.0 - 3.0 * np.arange(NUM_HEADS, dtype=np.float64) / NUM_HEADS))
    ij = np.abs(idx[:, None] - idx[None, :])
    mask3 = np.transpose(decay[:, None, None] * ij[None], (1, 0, 2)).reshape(SEQ, HS)

    fm = np.zeros((NUM_HEADS, EMBED_DIM), np.float32)
    for h in range(NUM_HEADS):
        fm[h, h * 16:(h + 1) * 16] = 1.0
    MS = np.repeat(fm, SEQ, axis=0)
    return (cos.astype(np.float32), sin.astype(np.float32), P,
            mask3.astype(np.float32), fm, MS)


_COS, _SIN, _P, _MASK3, _FM, _MS = _consts()


def _ln(x, g, b, eps=1e-5):
    mu = jnp.mean(x, axis=-1, keepdims=True)
    var = jnp.mean((x - mu) ** 2, axis=-1, keepdims=True)
    return (x - mu) * jax.lax.rsqrt(var + eps) * g + b


def _block_kernel(xp_ref, wq_ref, bq_ref, wk_ref, bk_ref, wv_ref, bv_ref,
                  g1_ref, be1_ref, w1_ref, b1_ref, w2_ref, b2_ref,
                  g2_ref, be2_ref, cos_ref, sin_ref, p_ref, mask_ref,
                  fm_ref, ms_ref, out_ref):
    f32 = jnp.float32
    bf16 = jnp.bfloat16
    xb = xp_ref[:]
    xb_bf = xb.astype(bf16)

    def mm(a, b, prefer=f32):
        return jax.lax.dot_general(a, b, (((1,), (0,)), ((), ())),
                                   preferred_element_type=prefer)

    def mm_nt(a, b, prefer=f32):
        return jax.lax.dot_general(a, b, (((1,), (1,)), ((), ())),
                                   preferred_element_type=prefer)

    cos = cos_ref[:]
    sin = sin_ref[:]
    P = p_ref[:]
    fm = fm_ref[:]

    def rot_bf(t_bf):
        tp = mm(t_bf, P).astype(bf16)
        t3 = t_bf.reshape(TBLK, SEQ, EMBED_DIM)
        tp3 = tp.reshape(TBLK, SEQ, EMBED_DIM)
        return (t3 * cos[None] + tp3 * sin[None]).reshape(ROWS, EMBED_DIM)

    q_bf = (mm(xb_bf, wq_ref[:]) + bq_ref[:]).astype(bf16)
    k_bf = (mm(xb_bf, wk_ref[:]) + bk_ref[:]).astype(bf16)
    v_bf = (mm(xb_bf, wv_ref[:]) + bv_ref[:]).astype(bf16)

    qr = rot_bf(q_bf)
    kr = rot_bf(k_bf)

    km = (kr.reshape(TBLK, 1, SEQ, EMBED_DIM) * fm[None, :, None, :]
          ).reshape(TBLK * HS, EMBED_DIM)
    vm = (v_bf.reshape(TBLK, 1, SEQ, EMBED_DIM) * fm[None, :, None, :]
          ).reshape(TBLK * HS, EMBED_DIM)

    mask3 = mask_ref[:]

    e_list = []
    for t in range(TBLK):
        s3 = mm_nt(qr[t * SEQ:(t + 1) * SEQ], km[t * HS:(t + 1) * HS])
        e_list.append(jnp.exp(s3 + mask3).astype(bf16))
    e_all = jnp.concatenate(e_list, axis=0)

    den = mm(e_all, ms_ref[:])

    o_list = []
    for t in range(TBLK):
        onum = mm(e_list[t], vm[t * HS:(t + 1) * HS])
        o_list.append(onum)
    attn = jnp.concatenate(o_list, axis=0) / den

    y = _ln(attn + xb, g1_ref[:], be1_ref[:])
    h1 = jnp.maximum(mm(y.astype(bf16), w1_ref[:]) + b1_ref[:], 0.0)
    ffn = mm(h1.astype(bf16), w2_ref[:]) + b2_ref[:]
    out_ref[:] = _ln(ffn + y, g2_ref[:], be2_ref[:])


@jax.jit
def kernel(x, Wq, bq, Wk, bk, Wv, bv, g1, be1, W1, b1, W2, b2, g2, be2):
    B, N, C = x.shape
    xp = jnp.transpose(x, (1, 0, 2)).reshape(N * B, C)
    bf16 = jnp.bfloat16

    grid = N // TBLK
    full = lambda shape: pl.BlockSpec(shape, lambda i: (0,) * len(shape))
    out = pl.pallas_call(
        _block_kernel,
        grid=(grid,),
        in_specs=[
            pl.BlockSpec((ROWS, C), lambda i: (i, 0)),
            full((C, C)), full((1, C)),
            full((C, C)), full((1, C)),
            full((C, C)), full((1, C)),
            full((1, C)), full((1, C)),
            full((C, 4 * C)), full((1, 4 * C)),
            full((4 * C, C)), full((1, C)),
            full((1, C)), full((1, C)),
            full((SEQ, C)), full((SEQ, C)), full((C, C)),
            full((SEQ, HS)), full((NUM_HEADS, C)), full((HS, C)),
        ],
        out_specs=pl.BlockSpec((ROWS, C), lambda i: (i, 0)),
        out_shape=jax.ShapeDtypeStruct((N * B, C), jnp.float32),
    )(xp, Wq.astype(bf16), bq.reshape(1, C), Wk.astype(bf16),
      bk.reshape(1, C), Wv.astype(bf16), bv.reshape(1, C),
      g1.reshape(1, C), be1.reshape(1, C),
      W1.astype(bf16), b1.reshape(1, 4 * C),
      W2.astype(bf16), b2.reshape(1, C), g2.reshape(1, C), be2.reshape(1, C),
      jnp.asarray(_COS, bf16), jnp.asarray(_SIN, bf16),
      jnp.asarray(_P, bf16), jnp.asarray(_MASK3),
      jnp.asarray(_FM, bf16), jnp.asarray(_MS, bf16))

    return out.reshape(N, B, C).transpose(1, 0, 2)


# speedup vs baseline: 1.5243x; 1.0881x over previous
import jax
import jax.numpy as jnp
import numpy as np
from jax.experimental import pallas as pl

EMBED_DIM = 128
NUM_HEADS = 8
HEAD_DIM = EMBED_DIM // NUM_HEADS
SEQ = 32
NTOK = 750
TBLK = 50
ROWS = TBLK * SEQ
HS = NUM_HEADS * SEQ


def _consts():
    half = HEAD_DIM // 2
    angle = 1.0 / 10000.0 ** np.linspace(0.0, 1.0, half)
    angle = np.repeat(angle, 2)
    angle_full = np.tile(angle, NUM_HEADS)
    idx = np.arange(SEQ, dtype=np.float64)
    sin = np.sin(idx[:, None] * angle_full[None, :])
    cos = np.cos(idx[:, None] * angle_full[None, :])

    P16 = np.zeros((HEAD_DIM, HEAD_DIM), np.float32)
    for o in range(half):
        P16[2 * o + 1, o] = -1.0
    for o in range(half, HEAD_DIM):
        P16[2 * (o - half), o] = 1.0
    P = np.zeros((EMBED_DIM, EMBED_DIM), np.float32)
    for h in range(NUM_HEADS):
        P[h * 16:(h + 1) * 16, h * 16:(h + 1) * 16] = P16

    decay = np.log(1.0 - 2.0 ** (-1.0 - 3.0 * np.arange(NUM_HEADS, dtype=np.float64) / NUM_HEADS))
    ij = np.abs(idx[:, None] - idx[None, :])
    mask3 = np.transpose(decay[:, None, None] * ij[None], (1, 0, 2)).reshape(SEQ, HS)

    fm = np.zeros((NUM_HEADS, EMBED_DIM), np.float32)
    for h in range(NUM_HEADS):
        fm[h, h * 16:(h + 1) * 16] = 1.0
    MS = np.repeat(fm, SEQ, axis=0)
    return (cos.astype(np.float32), sin.astype(np.float32), P,
            mask3.astype(np.float32), fm, MS)


_COS, _SIN, _P, _MASK3, _FM, _MS = _consts()


def _ln(x, g, b, eps=1e-5):
    mu = jnp.mean(x, axis=-1, keepdims=True)
    var = jnp.mean((x - mu) ** 2, axis=-1, keepdims=True)
    return (x - mu) * jax.lax.rsqrt(var + eps) * g + b


def _block_kernel(xp_ref, xbf_ref, wq_ref, bq_ref, wk_ref, bk_ref, wv_ref, bv_ref,
                  g1_ref, be1_ref, w1_ref, b1_ref, w2_ref, b2_ref,
                  g2_ref, be2_ref, cos_ref, sin_ref, p_ref, mask_ref,
                  fm_ref, ms_ref, out_ref):
    f32 = jnp.float32
    bf16 = jnp.bfloat16
    xb = xp_ref[:]
    xb_bf = xbf_ref[:]

    def mm(a, b, prefer=f32):
        return jax.lax.dot_general(a, b, (((1,), (0,)), ((), ())),
                                   preferred_element_type=prefer)

    def mm_nt(a, b, prefer=f32):
        return jax.lax.dot_general(a, b, (((1,), (1,)), ((), ())),
                                   preferred_element_type=prefer)

    cos = cos_ref[:]
    sin = sin_ref[:]
    P = p_ref[:]
    fm = fm_ref[:]

    def rot_bf(t_bf):
        tp = mm(t_bf, P).astype(bf16)
        t3 = t_bf.reshape(TBLK, SEQ, EMBED_DIM)
        tp3 = tp.reshape(TBLK, SEQ, EMBED_DIM)
        return (t3 * cos[None] + tp3 * sin[None]).reshape(ROWS, EMBED_DIM)

    q_bf = (mm(xb_bf, wq_ref[:]) + bq_ref[:]).astype(bf16)
    k_bf = (mm(xb_bf, wk_ref[:]) + bk_ref[:]).astype(bf16)
    v_bf = (mm(xb_bf, wv_ref[:]) + bv_ref[:]).astype(bf16)

    qr = rot_bf(q_bf)
    kr = rot_bf(k_bf)

    km = (kr.reshape(TBLK, 1, SEQ, EMBED_DIM) * fm[None, :, None, :]
          ).reshape(TBLK * HS, EMBED_DIM)
    vm = (v_bf.reshape(TBLK, 1, SEQ, EMBED_DIM) * fm[None, :, None, :]
          ).reshape(TBLK * HS, EMBED_DIM)

    mask3 = mask_ref[:]

    e_list = []
    for t in range(TBLK):
        s3 = mm_nt(qr[t * SEQ:(t + 1) * SEQ], km[t * HS:(t + 1) * HS])
        e_list.append(jnp.exp(s3 + mask3).astype(bf16))
    e_all = jnp.concatenate(e_list, axis=0)

    den = mm(e_all, ms_ref[:])

    o_list = []
    for t in range(TBLK):
        onum = mm(e_list[t], vm[t * HS:(t + 1) * HS])
        o_list.append(onum)
    attn = jnp.concatenate(o_list, axis=0) / den

    y = _ln(attn + xb, g1_ref[:], be1_ref[:])
    h1 = jnp.maximum(mm(y.astype(bf16), w1_ref[:]) + b1_ref[:], 0.0)
    ffn = mm(h1.astype(bf16), w2_ref[:]) + b2_ref[:]
    out_ref[:] = _ln(ffn + y, g2_ref[:], be2_ref[:])


@jax.jit
def kernel(x, Wq, bq, Wk, bk, Wv, bv, g1, be1, W1, b1, W2, b2, g2, be2):
    B, N, C = x.shape
    xp = jnp.transpose(x, (1, 0, 2)).reshape(N * B, C)
    bf16 = jnp.bfloat16

    grid = N // TBLK
    full = lambda shape: pl.BlockSpec(shape, lambda i: (0,) * len(shape))
    out = pl.pallas_call(
        _block_kernel,
        grid=(grid,),
        in_specs=[
            pl.BlockSpec((ROWS, C), lambda i: (i, 0)),
            pl.BlockSpec((ROWS, C), lambda i: (i, 0)),
            full((C, C)), full((1, C)),
            full((C, C)), full((1, C)),
            full((C, C)), full((1, C)),
            full((1, C)), full((1, C)),
            full((C, 4 * C)), full((1, 4 * C)),
            full((4 * C, C)), full((1, C)),
            full((1, C)), full((1, C)),
            full((SEQ, C)), full((SEQ, C)), full((C, C)),
            full((SEQ, HS)), full((NUM_HEADS, C)), full((HS, C)),
        ],
        out_specs=pl.BlockSpec((ROWS, C), lambda i: (i, 0)),
        out_shape=jax.ShapeDtypeStruct((N * B, C), jnp.float32),
    )(xp, xp.astype(bf16), Wq.astype(bf16), bq.reshape(1, C), Wk.astype(bf16),
      bk.reshape(1, C), Wv.astype(bf16), bv.reshape(1, C),
      g1.reshape(1, C), be1.reshape(1, C),
      W1.astype(bf16), b1.reshape(1, 4 * C),
      W2.astype(bf16), b2.reshape(1, C), g2.reshape(1, C), be2.reshape(1, C),
      jnp.asarray(_COS, bf16), jnp.asarray(_SIN, bf16),
      jnp.asarray(_P, bf16), jnp.asarray(_MASK3),
      jnp.asarray(_FM, bf16), jnp.asarray(_MS, bf16))

    return out.reshape(N, B, C).transpose(1, 0, 2)


# speedup vs baseline: 1.5548x; 1.0201x over previous
import jax
import jax.numpy as jnp
import numpy as np
from jax.experimental import pallas as pl

EMBED_DIM = 128
NUM_HEADS = 8
HEAD_DIM = EMBED_DIM // NUM_HEADS
SEQ = 32
NTOK = 750
TBLK = 75
ROWS = TBLK * SEQ
HS = NUM_HEADS * SEQ


def _consts():
    half = HEAD_DIM // 2
    angle = 1.0 / 10000.0 ** np.linspace(0.0, 1.0, half)
    angle = np.repeat(angle, 2)
    angle_full = np.tile(angle, NUM_HEADS)
    idx = np.arange(SEQ, dtype=np.float64)
    sin = np.sin(idx[:, None] * angle_full[None, :])
    cos = np.cos(idx[:, None] * angle_full[None, :])

    P16 = np.zeros((HEAD_DIM, HEAD_DIM), np.float32)
    for o in range(half):
        P16[2 * o + 1, o] = -1.0
    for o in range(half, HEAD_DIM):
        P16[2 * (o - half), o] = 1.0
    P = np.zeros((EMBED_DIM, EMBED_DIM), np.float32)
    for h in range(NUM_HEADS):
        P[h * 16:(h + 1) * 16, h * 16:(h + 1) * 16] = P16

    decay = np.log(1.0 - 2.0 ** (-1.0 - 3.0 * np.arange(NUM_HEADS, dtype=np.float64) / NUM_HEADS))
    ij = np.abs(idx[:, None] - idx[None, :])
    mask3 = np.transpose(decay[:, None, None] * ij[None], (1, 0, 2)).reshape(SEQ, HS)

    fm = np.zeros((NUM_HEADS, EMBED_DIM), np.float32)
    for h in range(NUM_HEADS):
        fm[h, h * 16:(h + 1) * 16] = 1.0
    MS = np.repeat(fm, SEQ, axis=0)
    return (cos.astype(np.float32), sin.astype(np.float32), P,
            mask3.astype(np.float32), fm, MS)


_COS, _SIN, _P, _MASK3, _FM, _MS = _consts()


def _ln(x, g, b, eps=1e-5):
    mu = jnp.mean(x, axis=-1, keepdims=True)
    var = jnp.mean((x - mu) ** 2, axis=-1, keepdims=True)
    return (x - mu) * jax.lax.rsqrt(var + eps) * g + b


def _block_kernel(xp_ref, xbf_ref, wq_ref, bq_ref, wk_ref, bk_ref, wv_ref, bv_ref,
                  g1_ref, be1_ref, w1_ref, b1_ref, w2_ref, b2_ref,
                  g2_ref, be2_ref, cos_ref, sin_ref, p_ref, mask_ref,
                  fm_ref, ms_ref, out_ref):
    f32 = jnp.float32
    bf16 = jnp.bfloat16
    xb = xp_ref[:]
    xb_bf = xbf_ref[:]

    def mm(a, b, prefer=f32):
        return jax.lax.dot_general(a, b, (((1,), (0,)), ((), ())),
                                   preferred_element_type=prefer)

    def mm_nt(a, b, prefer=f32):
        return jax.lax.dot_general(a, b, (((1,), (1,)), ((), ())),
                                   preferred_element_type=prefer)

    cos = cos_ref[:]
    sin = sin_ref[:]
    P = p_ref[:]
    fm = fm_ref[:]

    def rot_bf(t_bf):
        tp = mm(t_bf, P).astype(bf16)
        t3 = t_bf.reshape(TBLK, SEQ, EMBED_DIM)
        tp3 = tp.reshape(TBLK, SEQ, EMBED_DIM)
        return (t3 * cos[None] + tp3 * sin[None]).reshape(ROWS, EMBED_DIM)

    q_bf = (mm(xb_bf, wq_ref[:]) + bq_ref[:]).astype(bf16)
    k_bf = (mm(xb_bf, wk_ref[:]) + bk_ref[:]).astype(bf16)
    v_bf = (mm(xb_bf, wv_ref[:]) + bv_ref[:]).astype(bf16)

    qr = rot_bf(q_bf)
    kr = rot_bf(k_bf)

    km = (kr.reshape(TBLK, 1, SEQ, EMBED_DIM) * fm[None, :, None, :]
          ).reshape(TBLK * HS, EMBED_DIM)
    vm = (v_bf.reshape(TBLK, 1, SEQ, EMBED_DIM) * fm[None, :, None, :]
          ).reshape(TBLK * HS, EMBED_DIM)

    mask3 = mask_ref[:]

    e_list = []
    for t in range(TBLK):
        s3 = mm_nt(qr[t * SEQ:(t + 1) * SEQ], km[t * HS:(t + 1) * HS])
        e_list.append(jnp.exp(s3 + mask3).astype(bf16))
    e_all = jnp.concatenate(e_list, axis=0)

    den = mm(e_all, ms_ref[:])

    o_list = []
    for t in range(TBLK):
        onum = mm(e_list[t], vm[t * HS:(t + 1) * HS])
        o_list.append(onum)
    attn = jnp.concatenate(o_list, axis=0) / den

    y = _ln(attn + xb, g1_ref[:], be1_ref[:])
    h1 = jnp.maximum(mm(y.astype(bf16), w1_ref[:]) + b1_ref[:], 0.0)
    ffn = mm(h1.astype(bf16), w2_ref[:]) + b2_ref[:]
    out_ref[:] = _ln(ffn + y, g2_ref[:], be2_ref[:])


@jax.jit
def kernel(x, Wq, bq, Wk, bk, Wv, bv, g1, be1, W1, b1, W2, b2, g2, be2):
    B, N, C = x.shape
    xp = jnp.transpose(x, (1, 0, 2)).reshape(N * B, C)
    bf16 = jnp.bfloat16

    grid = N // TBLK
    full = lambda shape: pl.BlockSpec(shape, lambda i: (0,) * len(shape))
    out = pl.pallas_call(
        _block_kernel,
        grid=(grid,),
        in_specs=[
            pl.BlockSpec((ROWS, C), lambda i: (i, 0)),
            pl.BlockSpec((ROWS, C), lambda i: (i, 0)),
            full((C, C)), full((1, C)),
            full((C, C)), full((1, C)),
            full((C, C)), full((1, C)),
            full((1, C)), full((1, C)),
            full((C, 4 * C)), full((1, 4 * C)),
            full((4 * C, C)), full((1, C)),
            full((1, C)), full((1, C)),
            full((SEQ, C)), full((SEQ, C)), full((C, C)),
            full((SEQ, HS)), full((NUM_HEADS, C)), full((HS, C)),
        ],
        out_specs=pl.BlockSpec((ROWS, C), lambda i: (i, 0)),
        out_shape=jax.ShapeDtypeStruct((N * B, C), jnp.float32),
    )(xp, xp.astype(bf16), Wq.astype(bf16), bq.reshape(1, C), Wk.astype(bf16),
      bk.reshape(1, C), Wv.astype(bf16), bv.reshape(1, C),
      g1.reshape(1, C), be1.reshape(1, C),
      W1.astype(bf16), b1.reshape(1, 4 * C),
      W2.astype(bf16), b2.reshape(1, C), g2.reshape(1, C), be2.reshape(1, C),
      jnp.asarray(_COS, bf16), jnp.asarray(_SIN, bf16),
      jnp.asarray(_P, bf16), jnp.asarray(_MASK3),
      jnp.asarray(_FM, bf16), jnp.asarray(_MS, bf16))

    return out.reshape(N, B, C).transpose(1, 0, 2)
